# baseline (speedup 1.0000x reference)
import functools

import jax
import jax.numpy as jnp
from jax import lax
from jax.experimental import pallas as pl
from jax.experimental.pallas import tpu as pltpu
from jax.experimental.pallas import tpu_sc as plsc

_MAX_MASK_PCT = 0.15
_NUM_MASKS = 2
_B, _P, _D = 16, 2048, 1024
_NW = 32
_RPW = _B * _P // _NW
_BP = 256


def _segment_bounds(X_len):
    rk = jax.random.key(42)
    ka, kb = jax.random.split(rk)
    valid = X_len
    mml = jnp.floor(_MAX_MASK_PCT * valid.astype(jnp.float32)).astype(jnp.int32)
    vrep = jnp.repeat(valid, _NUM_MASKS)
    mrep = jnp.repeat(mml, _NUM_MASKS)
    n = _B * _NUM_MASKS
    t = jnp.floor(jax.random.uniform(ka, (n,)) * (mrep + 1).astype(jnp.float32)).astype(jnp.int32)
    max_start = jnp.clip(vrep - t + 1, 1, None)
    t0 = jnp.floor(jax.random.uniform(kb, (n,)) * max_start.astype(jnp.float32)).astype(jnp.int32)
    t1 = t0 + t
    return jnp.stack(
        [t0.reshape(_B, _NUM_MASKS), t1.reshape(_B, _NUM_MASKS)], axis=-1
    ).reshape(_B, 4)


_mesh = plsc.VectorSubcoreMesh(core_axis_name="c", subcore_axis_name="s")


@functools.partial(
    pl.kernel,
    mesh=_mesh,
    out_type=jax.ShapeDtypeStruct((_B * _P,), jnp.int32),
    scratch_types=[
        pltpu.VMEM((_RPW,), jnp.int32),
        pltpu.VMEM((16,), jnp.int32),
    ],
)
def _sc_mask_build(segs_hbm, mask_hbm, maskbuf, segs_v):
    wid = lax.axis_index("s") * 2 + lax.axis_index("c")
    base = wid * _RPW
    p0 = (wid % 2) * _RPW

    pltpu.sync_copy(segs_hbm.at[wid], segs_v)
    sv = segs_v[:]
    s0 = sv[0]
    e0 = sv[1]
    s1 = sv[2]
    e1 = sv[3]

    one16 = jnp.full((16,), 1, jnp.int32)
    zero16 = jnp.zeros((16,), jnp.int32)

    def mrow(i, c):
        p = p0 + i * 16 + lax.iota(jnp.int32, 16)
        m = ((p >= s0) & (p < e0)) | ((p >= s1) & (p < e1))
        maskbuf[pl.ds(i * 16, 16)] = jnp.where(m, one16, zero16)
        return c

    lax.fori_loop(0, _RPW // 16, mrow, 0)
    pltpu.sync_copy(maskbuf, mask_hbm.at[pl.ds(base, _RPW)])


def _tc_body(segs_ref, mval_ref, x_ref, o_ref):
    b = pl.program_id(0)
    j = pl.program_id(1)
    s0 = segs_ref[4 * b]
    e0 = segs_ref[4 * b + 1]
    s1 = segs_ref[4 * b + 2]
    e1 = segs_ref[4 * b + 3]
    lo = j * _BP
    hi = lo + _BP
    inside = ((lo >= s0) & (hi <= e0)) | ((lo >= s1) & (hi <= e1))
    clear0 = (hi <= s0) | (lo >= e0) | (e0 <= s0)
    clear1 = (hi <= s1) | (lo >= e1) | (e1 <= s1)
    untouched = clear0 & clear1

    @pl.when(untouched)
    def _():
        o_ref[...] = x_ref[...]

    @pl.when(inside)
    def _():
        o_ref[...] = jnp.full((1, _BP, _D), mval_ref[0], jnp.float32)

    @pl.when(jnp.logical_not(untouched | inside))
    def _():
        p = lo + lax.broadcasted_iota(jnp.int32, (1, _BP, 1), 1)
        m = ((p >= s0) & (p < e0)) | ((p >= s1) & (p < e1))
        o_ref[...] = jnp.where(m, mval_ref[0], x_ref[...])


_tc_masked_copy = pl.pallas_call(
    _tc_body,
    grid=(_B, _P // _BP),
    in_specs=[
        pl.BlockSpec(memory_space=pltpu.SMEM),
        pl.BlockSpec(memory_space=pltpu.SMEM),
        pl.BlockSpec((1, _BP, _D), lambda b, j: (b, j, 0)),
    ],
    out_specs=pl.BlockSpec((1, _BP, _D), lambda b, j: (b, j, 0)),
    out_shape=jax.ShapeDtypeStruct((_B, _P, _D), jnp.float32),
)


def kernel(X, X_len, mask_value):
    segs = _segment_bounds(X_len)
    segs_w = jnp.repeat(jnp.pad(segs, ((0, 0), (0, 12))), _NW // _B, axis=0)
    mask_i32 = _sc_mask_build(segs_w)
    out = _tc_masked_copy(segs.reshape(_B * 4), mask_value, X)
    return out, mask_i32.reshape(_B, _P) != 0


# speedup vs baseline: 1.4394x; 1.4394x over previous
import functools

import jax
import jax.numpy as jnp
from jax import lax
from jax.experimental import pallas as pl
from jax.experimental.pallas import tpu as pltpu
from jax.experimental.pallas import tpu_sc as plsc

_MAX_MASK_PCT = 0.15
_NUM_MASKS = 2
_B, _P, _D = 16, 2048, 1024
_NW = 32
_RPW = _B * _P // _NW
_BP = 2048


def _segment_bounds(X_len):
    rk = jax.random.key(42)
    ka, kb = jax.random.split(rk)
    valid = X_len
    mml = jnp.floor(_MAX_MASK_PCT * valid.astype(jnp.float32)).astype(jnp.int32)
    vrep = jnp.repeat(valid, _NUM_MASKS)
    mrep = jnp.repeat(mml, _NUM_MASKS)
    n = _B * _NUM_MASKS
    t = jnp.floor(jax.random.uniform(ka, (n,)) * (mrep + 1).astype(jnp.float32)).astype(jnp.int32)
    max_start = jnp.clip(vrep - t + 1, 1, None)
    t0 = jnp.floor(jax.random.uniform(kb, (n,)) * max_start.astype(jnp.float32)).astype(jnp.int32)
    t1 = t0 + t
    return jnp.stack(
        [t0.reshape(_B, _NUM_MASKS), t1.reshape(_B, _NUM_MASKS)], axis=-1
    ).reshape(_B, 4)


_mesh = plsc.VectorSubcoreMesh(core_axis_name="c", subcore_axis_name="s")


@functools.partial(
    pl.kernel,
    mesh=_mesh,
    out_type=jax.ShapeDtypeStruct((_B * _P,), jnp.int32),
    scratch_types=[
        pltpu.VMEM((_RPW,), jnp.int32),
        pltpu.VMEM((16,), jnp.int32),
    ],
)
def _sc_mask_build(segs_hbm, mask_hbm, maskbuf, segs_v):
    wid = lax.axis_index("s") * 2 + lax.axis_index("c")
    base = wid * _RPW
    p0 = (wid % 2) * _RPW

    pltpu.sync_copy(segs_hbm.at[wid], segs_v)
    sv = segs_v[:]
    s0 = sv[0]
    e0 = sv[1]
    s1 = sv[2]
    e1 = sv[3]

    one16 = jnp.full((16,), 1, jnp.int32)
    zero16 = jnp.zeros((16,), jnp.int32)

    def mrow(i, c):
        p = p0 + i * 16 + lax.iota(jnp.int32, 16)
        m = ((p >= s0) & (p < e0)) | ((p >= s1) & (p < e1))
        maskbuf[pl.ds(i * 16, 16)] = jnp.where(m, one16, zero16)
        return c

    lax.fori_loop(0, _RPW // 16, mrow, 0)
    pltpu.sync_copy(maskbuf, mask_hbm.at[pl.ds(base, _RPW)])


def _tc_body(segs_ref, mval_ref, x_ref, o_ref):
    b = pl.program_id(0)
    j = pl.program_id(1)
    s0 = segs_ref[4 * b]
    e0 = segs_ref[4 * b + 1]
    s1 = segs_ref[4 * b + 2]
    e1 = segs_ref[4 * b + 3]
    p = j * _BP + lax.broadcasted_iota(jnp.int32, (1, _BP, 1), 1)
    m = ((p >= s0) & (p < e0)) | ((p >= s1) & (p < e1))
    o_ref[...] = jnp.where(m, mval_ref[0], x_ref[...])


_tc_masked_copy = pl.pallas_call(
    _tc_body,
    grid=(_B, _P // _BP),
    in_specs=[
        pl.BlockSpec(memory_space=pltpu.SMEM),
        pl.BlockSpec(memory_space=pltpu.SMEM),
        pl.BlockSpec((1, _BP, _D), lambda b, j: (b, j, 0)),
    ],
    out_specs=pl.BlockSpec((1, _BP, _D), lambda b, j: (b, j, 0)),
    out_shape=jax.ShapeDtypeStruct((_B, _P, _D), jnp.float32),
)


def kernel(X, X_len, mask_value):
    segs = _segment_bounds(X_len)
    segs_w = jnp.repeat(jnp.pad(segs, ((0, 0), (0, 12))), _NW // _B, axis=0)
    mask_i32 = _sc_mask_build(segs_w)
    out = _tc_masked_copy(segs.reshape(_B * 4), mask_value, X)
    return out, mask_i32.reshape(_B, _P) != 0
